# baseline (speedup 1.0000x reference)
import functools

import jax
import jax.numpy as jnp
from jax import lax
from jax.experimental import pallas as pl
from jax.experimental.pallas import tpu as pltpu
from jax.experimental.pallas import tpu_sc as plsc

B, L, V, D = 1024, 200, 100000, 64
LANE = 16
G = D // LANE
NC, NS = 2, 16
NW = NC * NS
RPW = B // NW
LCH = 100
NCH = L // LCH
NBUF = 2
DP = 128


def _bag_kernel(tok2, wt, bias):
    mesh = plsc.VectorSubcoreMesh(core_axis_name="c", subcore_axis_name="s")

    @functools.partial(
        pl.kernel,
        out_type=jax.ShapeDtypeStruct((B, D), jnp.float32),
        mesh=mesh,
        scratch_types=[
            pltpu.VMEM((RPW * NCH, LCH), jnp.int32),
            pltpu.VMEM((NBUF, L, DP), jnp.float32),
            pltpu.VMEM((RPW, D), jnp.float32),
            pltpu.VMEM((D,), jnp.float32),
            pltpu.SemaphoreType.DMA,
            pltpu.SemaphoreType.DMA,
        ],
    )
    def k(tok_hbm, wt_hbm, b_hbm, out_hbm, idx_v, rows_v, out_v, bias_v,
          sem0, sem1):
        sems = (sem0, sem1)
        wid = lax.axis_index("s") * NC + lax.axis_index("c")
        base = wid * RPW
        pltpu.sync_copy(b_hbm, bias_v)
        pltpu.sync_copy(tok_hbm.at[pl.ds(base * NCH, RPW * NCH)], idx_v)

        def issue(i, s):
            for c in range(NCH):
                pltpu.async_copy(
                    wt_hbm.at[idx_v.at[i * NCH + c]],
                    rows_v.at[s, pl.ds(c * LCH, LCH)],
                    sems[s],
                )

        def drain(s):
            for c in range(NCH):
                pltpu.make_async_copy(
                    wt_hbm.at[idx_v.at[c]],
                    rows_v.at[s, pl.ds(c * LCH, LCH)],
                    sems[s],
                ).wait()

        for s in range(NBUF):
            issue(s, s)

        def pair_body(p, carry):
            for s in range(NBUF):
                i = p * NBUF + s
                drain(s)

                @pl.when(i + NBUF < RPW)
                def _():
                    issue(i + NBUF, s)

                def tok_body(j, accs):
                    return tuple(
                        a + rows_v[s, j, pl.ds(g * LANE, LANE)]
                        for g, a in enumerate(accs)
                    )

                accs = tuple(bias_v[pl.ds(g * LANE, LANE)] for g in range(G))
                accs = lax.fori_loop(0, L, tok_body, accs, unroll=8)
                for g in range(G):
                    out_v[i, pl.ds(g * LANE, LANE)] = accs[g]
            return carry

        lax.fori_loop(0, RPW // NBUF, pair_body, 0)
        pltpu.sync_copy(out_v, out_hbm.at[pl.ds(base, RPW)])

    return k(tok2, wt, bias)


def kernel(token_ids, W, b):
    tok2 = token_ids.astype(jnp.int32).reshape(B * NCH, LCH)
    wt = jnp.pad(W.T, ((0, 0), (0, DP - D)))
    out = _bag_kernel(tok2, wt, b)
    return out[:, None, :]


# speedup vs baseline: 1.1962x; 1.1962x over previous
import functools

import jax
import jax.numpy as jnp
from jax import lax
from jax.experimental import pallas as pl
from jax.experimental.pallas import tpu as pltpu
from jax.experimental.pallas import tpu_sc as plsc

B, L, V, D = 1024, 200, 100000, 64
LANE = 16
NC, NS = 2, 16
NW = NC * NS
NCHK = B // (4 * LANE)
NBUF = 2
LH = L // 2


def _pair_kernel(ids3, wpacked):
    mesh = plsc.VectorSubcoreMesh(core_axis_name="c", subcore_axis_name="s")

    @functools.partial(
        pl.kernel,
        out_type=jax.ShapeDtypeStruct((D, B), jnp.float32),
        mesh=mesh,
        compiler_params=pltpu.CompilerParams(
            needs_layout_passes=False, use_tc_tiling_on_sc=False),
        scratch_types=[
            pltpu.VMEM((V,), jnp.int32),
            pltpu.VMEM((NBUF, LH, 64), jnp.int32),
            pltpu.VMEM((B,), jnp.float32),
            pltpu.VMEM((B,), jnp.float32),
            pltpu.SemaphoreType.DMA,
            pltpu.SemaphoreType.DMA,
        ],
    )
    def k(ids_hbm, wp_hbm, out_hbm, wrow_v, chunk_v, out0_v, out1_v,
          sem0, sem1):
        sems = (sem0, sem1)
        wid = lax.axis_index("s") * NC + lax.axis_index("c")
        pltpu.sync_copy(wp_hbm.at[wid], wrow_v)

        def issue(step, s):
            pltpu.async_copy(
                ids_hbm.at[pl.ds(step * LH, LH)], chunk_v.at[s], sems[s])

        nsteps = NCHK * 2
        issue(0, 0)
        for c in range(NCHK):
            accs = tuple(jnp.zeros((LANE,), jnp.float32) for _ in range(8))
            for h in range(2):
                step = c * 2 + h
                s = step % NBUF
                if step + 1 < nsteps:
                    issue(step + 1, (step + 1) % NBUF)
                pltpu.make_async_copy(
                    ids_hbm.at[pl.ds(0, LH)], chunk_v.at[s], sems[s]).wait()

                def jbody(j, accs):
                    new = []
                    for sg in range(4):
                        ids16 = chunk_v[s, j, pl.ds(sg * LANE, LANE)]
                        g = plsc.load_gather(wrow_v, [ids16])
                        v0, v1 = plsc.unpack(
                            plsc.bitcast(g, jnp.bfloat16),
                            format=plsc.PackFormat.INTERLEAVED)
                        new.append(accs[2 * sg] + v0)
                        new.append(accs[2 * sg + 1] + v1)
                    return tuple(new)

                accs = lax.fori_loop(0, LH, jbody, accs, unroll=2)
            for sg in range(4):
                out0_v[pl.ds(c * 64 + sg * LANE, LANE)] = accs[2 * sg]
                out1_v[pl.ds(c * 64 + sg * LANE, LANE)] = accs[2 * sg + 1]

        pltpu.sync_copy(out0_v, out_hbm.at[wid])
        pltpu.sync_copy(out1_v, out_hbm.at[wid + NW])

    return k(ids3, wpacked)


def kernel(token_ids, W, b):
    ids3 = (token_ids.astype(jnp.int32)
            .reshape(NCHK, 4 * LANE, L)
            .transpose(0, 2, 1)
            .reshape(NCHK * L, 4 * LANE))
    lo = lax.bitcast_convert_type(
        W[:NW].astype(jnp.bfloat16), jnp.uint16).astype(jnp.uint32)
    hi = lax.bitcast_convert_type(
        W[NW:].astype(jnp.bfloat16), jnp.uint16).astype(jnp.uint32)
    wpacked = lax.bitcast_convert_type(lo | (hi << 16), jnp.int32)
    out_t = _pair_kernel(ids3, wpacked)
    return (out_t.T + b[None, :])[:, None, :]
